# baseline (speedup 1.0000x reference)
import functools

import jax
import jax.numpy as jnp
from jax import lax
from jax.experimental import pallas as pl
from jax.experimental.pallas import tpu as pltpu
from jax.experimental.pallas import tpu_sc as plsc

NN = 10000
EE = 320000
DD = 128
RR = 8
RN = RR * NN

CE = 6400
NCHUNK = EE // CE
NCORES = 2
NSUB = 16
NW = NCORES * NSUB
DPT = DD // NW
NCNT = 25
CNT_CHUNKS = NCHUNK // NCNT
NPAIR = NCHUNK // 2
SRC_BITS = 14
SRC_MASK = (1 << SRC_BITS) - 1


def _sc_body(make_counts, *refs):
    if make_counts:
        (xt, comb2, s_out, cnt_out, xrow, acc,
         cbuf0, cbuf1, sem0, sem1) = refs
    else:
        (xt, comb2, s_out, xrow, acc,
         cbuf0, cbuf1, sem0, sem1) = refs
        cnt_out = None

    c = lax.axis_index("c")
    s = lax.axis_index("s")
    wid = s * NCORES + c
    slots = ((cbuf0, sem0), (cbuf1, sem1))

    def zero_acc():
        @plsc.parallel_loop(0, RN // 16, unroll=8)
        def _(i):
            acc[pl.ds(i * 16, 16)] = jnp.zeros((16,), jnp.float32)

    def start_load(ci, slot):
        pltpu.async_copy(comb2.at[ci], slot[0], slot[1])

    def wait_load(slot):
        pltpu.make_async_copy(comb2.at[0], slot[0], slot[1]).wait()

    def process_chunk(slot):
        cb = slot[0]

        @plsc.parallel_loop(0, CE // 16, unroll=32)
        def _(j):
            c16 = cb[pl.ds(j * 16, 16)]
            s16 = jnp.bitwise_and(c16, SRC_MASK)
            a16 = jnp.right_shift(c16, SRC_BITS)
            v = plsc.load_gather(xrow, [s16])
            plsc.addupdate_scatter(acc, [a16], v)

    for k in range(DPT):
        d = wid * DPT + k
        start_load(0, slots[0])
        pltpu.sync_copy(xt.at[d], xrow)
        zero_acc()

        def pair(i, carry):
            ci = 2 * i
            start_load(ci + 1, slots[1])
            wait_load(slots[0])
            process_chunk(slots[0])

            @pl.when(i < NPAIR - 1)
            def _():
                start_load(ci + 2, slots[0])
            wait_load(slots[1])
            process_chunk(slots[1])
            return carry
        lax.fori_loop(0, NPAIR, pair, 0)

        for r in range(RR):
            pltpu.async_copy(acc.at[pl.ds(r * NN, NN)], s_out.at[r, d],
                             sem0)
        for r in range(RR):
            pltpu.make_async_copy(acc.at[pl.ds(r * NN, NN)], s_out.at[r, d],
                                  sem0).wait()

    if make_counts:
        @pl.when(wid < NCNT)
        def _():
            zero_acc()
            for j in range(CNT_CHUNKS):
                pltpu.sync_copy(comb2.at[wid * CNT_CHUNKS + j], cbuf0)

                @plsc.parallel_loop(0, CE // 16, unroll=16)
                def _(jj):
                    c16 = cbuf0[pl.ds(jj * 16, 16)]
                    a16 = jnp.right_shift(c16, SRC_BITS)
                    plsc.addupdate_scatter(acc, [a16],
                                           jnp.ones((16,), jnp.float32))
            for r in range(RR):
                pltpu.sync_copy(acc.at[pl.ds(r * NN, NN)], cnt_out.at[r, wid])


def _make_sc_layer(make_counts):
    out_type = [jax.ShapeDtypeStruct((RR, DD, NN), jnp.float32)]
    if make_counts:
        out_type.append(jax.ShapeDtypeStruct((RR, NCNT, NN), jnp.float32))
    mesh = plsc.VectorSubcoreMesh(core_axis_name="c", subcore_axis_name="s")
    return pl.kernel(
        functools.partial(_sc_body, make_counts),
        out_type=tuple(out_type),
        mesh=mesh,
        compiler_params=pltpu.CompilerParams(
            needs_layout_passes=False, use_tc_tiling_on_sc=False),
        scratch_types=[
            pltpu.VMEM((NN,), jnp.float32),
            pltpu.VMEM((RN,), jnp.float32),
            pltpu.VMEM((CE,), jnp.int32),
            pltpu.VMEM((CE,), jnp.int32),
            pltpu.SemaphoreType.DMA,
            pltpu.SemaphoreType.DMA,
        ],
    )


_sc_layer_with_counts = _make_sc_layer(True)
_sc_layer = _make_sc_layer(False)


def _tc_body(final, xt_ref, s_ref, cnt_ref, wrelT_ref, wrootT_ref, b_ref,
             out_ref, *rest):
    r = pl.program_id(0)
    if final:
        acc_ref, = rest
        invc = cnt_ref[0]
    else:
        invc_out_ref, acc_ref = rest
        cnt_r = jnp.sum(cnt_ref[0], axis=0, keepdims=True)
        invc = 1.0 / jnp.maximum(cnt_r, 1.0)
        invc_out_ref[0] = invc
    m = s_ref[0] * invc
    part = jnp.dot(wrelT_ref[0], m,
                   preferred_element_type=jnp.float32)

    @pl.when(r == 0)
    def _():
        root = jnp.dot(wrootT_ref[...], xt_ref[...],
                       preferred_element_type=jnp.float32)
        acc_ref[...] = root + b_ref[...]

    acc_ref[...] += part

    @pl.when(r == RR - 1)
    def _():
        res = jnp.maximum(acc_ref[...], 0.0)
        if final:
            out_ref[...] = res.T
        else:
            out_ref[...] = res


def _make_tc_layer(final):
    if final:
        out_specs = pl.BlockSpec((NN, DD), lambda r: (0, 0))
        out_shape = jax.ShapeDtypeStruct((NN, DD), jnp.float32)
        cnt_spec = pl.BlockSpec((1, 1, NN), lambda r: (r, 0, 0))
    else:
        out_specs = (
            pl.BlockSpec((DD, NN), lambda r: (0, 0)),
            pl.BlockSpec((1, 1, NN), lambda r: (r, 0, 0)),
        )
        out_shape = (
            jax.ShapeDtypeStruct((DD, NN), jnp.float32),
            jax.ShapeDtypeStruct((RR, 1, NN), jnp.float32),
        )
        cnt_spec = pl.BlockSpec((1, NCNT, NN), lambda r: (r, 0, 0))
    return pl.pallas_call(
        functools.partial(_tc_body, final),
        grid=(RR,),
        in_specs=[
            pl.BlockSpec((DD, NN), lambda r: (0, 0)),
            pl.BlockSpec((1, DD, NN), lambda r: (r, 0, 0)),
            cnt_spec,
            pl.BlockSpec((1, DD, DD), lambda r: (r, 0, 0)),
            pl.BlockSpec((DD, DD), lambda r: (0, 0)),
            pl.BlockSpec((DD, 1), lambda r: (0, 0)),
        ],
        out_specs=out_specs,
        out_shape=out_shape,
        scratch_shapes=[pltpu.VMEM((DD, NN), jnp.float32)],
    )


_tc_mid = _make_tc_layer(False)
_tc_final = _make_tc_layer(True)


def _tr_body(x_ref, o_ref):
    o_ref[...] = x_ref[...].T


_transpose_in = pl.pallas_call(
    _tr_body,
    in_specs=[pl.BlockSpec((NN, DD), lambda: (0, 0))],
    out_specs=pl.BlockSpec((DD, NN), lambda: (0, 0)),
    out_shape=jax.ShapeDtypeStruct((DD, NN), jnp.float32),
)


def kernel(x, edge_index, edge_type, Wrel0, Wroot0, b0, Wrel1, Wroot1, b1):
    src = edge_index[0].astype(jnp.int32)
    dst = edge_index[1].astype(jnp.int32)
    et = edge_type.astype(jnp.int32)
    aidx = et * NN + dst
    comb2 = ((aidx << SRC_BITS) | src).reshape(NCHUNK, CE)

    wrel0T = Wrel0.transpose(0, 2, 1)
    wrel1T = Wrel1.transpose(0, 2, 1)
    wroot0T = Wroot0.T
    wroot1T = Wroot1.T
    b0c = b0.reshape(DD, 1)
    b1c = b1.reshape(DD, 1)

    xt = _transpose_in(x)
    s1, cnt = _sc_layer_with_counts(xt, comb2)
    ht, invc = _tc_mid(xt, s1, cnt, wrel0T, wroot0T, b0c)
    (s2,) = _sc_layer(ht, comb2)
    out = _tc_final(ht, s2, invc, wrel1T, wroot1T, b1c)
    return out


# speedup vs baseline: 1.0889x; 1.0889x over previous
import functools

import jax
import jax.numpy as jnp
from jax import lax
from jax.experimental import pallas as pl
from jax.experimental.pallas import tpu as pltpu
from jax.experimental.pallas import tpu_sc as plsc

NN = 10000
EE = 320000
DD = 128
RR = 8
RN = RR * NN

CE = 6400
NCHUNK = EE // CE
NCORES = 2
NSUB = 16
NW = NCORES * NSUB
DPT = DD // NW
NCNT = 25
CNT_CHUNKS = NCHUNK // NCNT
NPAIR = NCHUNK // 2
SRC_BITS = 14
SRC_MASK = (1 << SRC_BITS) - 1


def _sc_body(make_counts, *refs):
    if make_counts:
        (xt, comb2, s_out, cnt_out, xrow, acc,
         cbuf0, cbuf1, sem0, sem1) = refs
    else:
        (xt, comb2, s_out, xrow, acc,
         cbuf0, cbuf1, sem0, sem1) = refs
        cnt_out = None

    c = lax.axis_index("c")
    s = lax.axis_index("s")
    wid = s * NCORES + c
    slots = ((cbuf0, sem0), (cbuf1, sem1))

    def zero_acc():
        @plsc.parallel_loop(0, RN // 16, unroll=8)
        def _(i):
            acc[pl.ds(i * 16, 16)] = jnp.zeros((16,), jnp.float32)

    def start_load(ci, slot):
        pltpu.async_copy(comb2.at[ci], slot[0], slot[1])

    def wait_load(slot):
        pltpu.make_async_copy(comb2.at[0], slot[0], slot[1]).wait()

    def process_chunk(slot):
        cb = slot[0]

        @plsc.parallel_loop(0, CE // 16, unroll=16)
        def _(j):
            c16 = cb[pl.ds(j * 16, 16)]
            s16 = jnp.bitwise_and(c16, SRC_MASK)
            a16 = jnp.right_shift(c16, SRC_BITS)
            v = plsc.load_gather(xrow, [s16])
            plsc.addupdate_scatter(acc, [a16], v)

    for k in range(DPT):
        d = wid * DPT + k
        start_load(0, slots[0])
        pltpu.sync_copy(xt.at[d], xrow)
        zero_acc()

        def pair(i, carry):
            ci = 2 * i
            start_load(ci + 1, slots[1])
            wait_load(slots[0])
            process_chunk(slots[0])

            @pl.when(i < NPAIR - 1)
            def _():
                start_load(ci + 2, slots[0])
            wait_load(slots[1])
            process_chunk(slots[1])
            return carry
        lax.fori_loop(0, NPAIR, pair, 0)

        for r in range(RR):
            pltpu.async_copy(acc.at[pl.ds(r * NN, NN)], s_out.at[r, d],
                             sem0)
        for r in range(RR):
            pltpu.make_async_copy(acc.at[pl.ds(r * NN, NN)], s_out.at[r, d],
                                  sem0).wait()

    if make_counts:
        @pl.when(wid < NCNT)
        def _():
            zero_acc()
            for j in range(CNT_CHUNKS):
                pltpu.sync_copy(comb2.at[wid * CNT_CHUNKS + j], cbuf0)

                @plsc.parallel_loop(0, CE // 16, unroll=16)
                def _(jj):
                    c16 = cbuf0[pl.ds(jj * 16, 16)]
                    a16 = jnp.right_shift(c16, SRC_BITS)
                    plsc.addupdate_scatter(acc, [a16],
                                           jnp.ones((16,), jnp.float32))
            for r in range(RR):
                pltpu.sync_copy(acc.at[pl.ds(r * NN, NN)], cnt_out.at[r, wid])


def _make_sc_layer(make_counts):
    out_type = [jax.ShapeDtypeStruct((RR, DD, NN), jnp.float32)]
    if make_counts:
        out_type.append(jax.ShapeDtypeStruct((RR, NCNT, NN), jnp.float32))
    mesh = plsc.VectorSubcoreMesh(core_axis_name="c", subcore_axis_name="s")
    return pl.kernel(
        functools.partial(_sc_body, make_counts),
        out_type=tuple(out_type),
        mesh=mesh,
        compiler_params=pltpu.CompilerParams(
            needs_layout_passes=False, use_tc_tiling_on_sc=False),
        scratch_types=[
            pltpu.VMEM((NN,), jnp.float32),
            pltpu.VMEM((RN,), jnp.float32),
            pltpu.VMEM((CE,), jnp.int32),
            pltpu.VMEM((CE,), jnp.int32),
            pltpu.SemaphoreType.DMA,
            pltpu.SemaphoreType.DMA,
        ],
    )


_sc_layer_with_counts = _make_sc_layer(True)
_sc_layer = _make_sc_layer(False)


def _tc_body(final, xt_ref, s_ref, cnt_ref, wrelT_ref, wrootT_ref, b_ref,
             out_ref, *rest):
    r = pl.program_id(0)
    if final:
        acc_ref, = rest
        invc = cnt_ref[0]
    else:
        invc_out_ref, acc_ref = rest
        cnt_r = jnp.sum(cnt_ref[0], axis=0, keepdims=True)
        invc = 1.0 / jnp.maximum(cnt_r, 1.0)
        invc_out_ref[0] = invc
    m = s_ref[0] * invc
    part = jnp.dot(wrelT_ref[0], m,
                   preferred_element_type=jnp.float32)

    @pl.when(r == 0)
    def _():
        root = jnp.dot(wrootT_ref[...], xt_ref[...],
                       preferred_element_type=jnp.float32)
        acc_ref[...] = root + b_ref[...]

    acc_ref[...] += part

    @pl.when(r == RR - 1)
    def _():
        res = jnp.maximum(acc_ref[...], 0.0)
        if final:
            out_ref[...] = res.T
        else:
            out_ref[...] = res


def _make_tc_layer(final):
    if final:
        out_specs = pl.BlockSpec((NN, DD), lambda r: (0, 0))
        out_shape = jax.ShapeDtypeStruct((NN, DD), jnp.float32)
        cnt_spec = pl.BlockSpec((1, 1, NN), lambda r: (r, 0, 0))
    else:
        out_specs = (
            pl.BlockSpec((DD, NN), lambda r: (0, 0)),
            pl.BlockSpec((1, 1, NN), lambda r: (r, 0, 0)),
        )
        out_shape = (
            jax.ShapeDtypeStruct((DD, NN), jnp.float32),
            jax.ShapeDtypeStruct((RR, 1, NN), jnp.float32),
        )
        cnt_spec = pl.BlockSpec((1, NCNT, NN), lambda r: (r, 0, 0))
    return pl.pallas_call(
        functools.partial(_tc_body, final),
        grid=(RR,),
        in_specs=[
            pl.BlockSpec((DD, NN), lambda r: (0, 0)),
            pl.BlockSpec((1, DD, NN), lambda r: (r, 0, 0)),
            cnt_spec,
            pl.BlockSpec((1, DD, DD), lambda r: (r, 0, 0)),
            pl.BlockSpec((DD, DD), lambda r: (0, 0)),
            pl.BlockSpec((DD, 1), lambda r: (0, 0)),
        ],
        out_specs=out_specs,
        out_shape=out_shape,
        scratch_shapes=[pltpu.VMEM((DD, NN), jnp.float32)],
    )


_tc_mid = _make_tc_layer(False)
_tc_final = _make_tc_layer(True)


def _tr_body(x_ref, o_ref):
    o_ref[...] = x_ref[...].T


_transpose_in = pl.pallas_call(
    _tr_body,
    in_specs=[pl.BlockSpec((NN, DD), lambda: (0, 0))],
    out_specs=pl.BlockSpec((DD, NN), lambda: (0, 0)),
    out_shape=jax.ShapeDtypeStruct((DD, NN), jnp.float32),
)


def kernel(x, edge_index, edge_type, Wrel0, Wroot0, b0, Wrel1, Wroot1, b1):
    src = edge_index[0].astype(jnp.int32)
    dst = edge_index[1].astype(jnp.int32)
    et = edge_type.astype(jnp.int32)
    aidx = et * NN + dst
    comb2 = ((aidx << SRC_BITS) | src).reshape(NCHUNK, CE)

    wrel0T = Wrel0.transpose(0, 2, 1)
    wrel1T = Wrel1.transpose(0, 2, 1)
    wroot0T = Wroot0.T
    wroot1T = Wroot1.T
    b0c = b0.reshape(DD, 1)
    b1c = b1.reshape(DD, 1)

    xt = _transpose_in(x)
    s1, cnt = _sc_layer_with_counts(xt, comb2)
    ht, invc = _tc_mid(xt, s1, cnt, wrel0T, wroot0T, b0c)
    (s2,) = _sc_layer(ht, comb2)
    out = _tc_final(ht, s2, invc, wrel1T, wroot1T, b1c)
    return out


# speedup vs baseline: 1.1091x; 1.0186x over previous
import functools

import jax
import jax.numpy as jnp
from jax import lax
from jax.experimental import pallas as pl
from jax.experimental.pallas import tpu as pltpu
from jax.experimental.pallas import tpu_sc as plsc

NN = 10000
EE = 320000
DD = 128
RR = 8
RN = RR * NN

CE = 6400
NCHUNK = EE // CE
NCORES = 2
NSUB = 16
NW = NCORES * NSUB
DPT = DD // NW
NCNT = 25
CNT_CHUNKS = NCHUNK // NCNT
NPAIR = NCHUNK // 2
SRC_BITS = 14
SRC_MASK = (1 << SRC_BITS) - 1


def _sc_body(make_counts, *refs):
    if make_counts:
        (xt, comb2, s_out, cnt_out, xrowA, xrowB, acc,
         cbuf0, cbuf1, sem0, sem1, semx, semw) = refs
    else:
        (xt, comb2, s_out, xrowA, xrowB, acc,
         cbuf0, cbuf1, sem0, sem1, semx, semw) = refs
        cnt_out = None

    c = lax.axis_index("c")
    s = lax.axis_index("s")
    wid = s * NCORES + c
    slots = ((cbuf0, sem0), (cbuf1, sem1))

    def zero_acc():
        @plsc.parallel_loop(0, RN // 16, unroll=8)
        def _(i):
            acc[pl.ds(i * 16, 16)] = jnp.zeros((16,), jnp.float32)

    def start_load(ci, slot):
        pltpu.async_copy(comb2.at[ci], slot[0], slot[1])

    def wait_load(slot):
        pltpu.make_async_copy(comb2.at[0], slot[0], slot[1]).wait()

    def process_chunk(slot, xrow):
        cb = slot[0]

        @plsc.parallel_loop(0, CE // 16, unroll=16)
        def _(j):
            c16 = cb[pl.ds(j * 16, 16)]
            s16 = jnp.bitwise_and(c16, SRC_MASK)
            a16 = jnp.right_shift(c16, SRC_BITS)
            v = plsc.load_gather(xrow, [s16])
            plsc.addupdate_scatter(acc, [a16], v)

    def zero_seg(r):
        @plsc.parallel_loop(0, NN // 16, unroll=8)
        def _(i):
            acc[pl.ds(r * NN + i * 16, 16)] = jnp.zeros((16,), jnp.float32)

    xrows = (xrowA, xrowB)
    pltpu.async_copy(xt.at[wid * DPT], xrowA, semx)
    for k in range(DPT):
        d = wid * DPT + k
        xrow = xrows[k % 2]
        start_load(0, slots[0])
        pltpu.make_async_copy(xt.at[0], xrow, semx).wait()
        if k + 1 < DPT:
            pltpu.async_copy(xt.at[d + 1], xrows[(k + 1) % 2], semx)
        if k == 0:
            zero_acc()
        else:
            for r in range(RR):
                pltpu.make_async_copy(acc.at[pl.ds(r * NN, NN)],
                                      s_out.at[r, d - 1], semw).wait()
                zero_seg(r)

        def pair(i, carry):
            ci = 2 * i
            start_load(ci + 1, slots[1])
            wait_load(slots[0])
            process_chunk(slots[0], xrow)

            @pl.when(i < NPAIR - 1)
            def _():
                start_load(ci + 2, slots[0])
            wait_load(slots[1])
            process_chunk(slots[1], xrow)
            return carry
        lax.fori_loop(0, NPAIR, pair, 0)

        for r in range(RR):
            pltpu.async_copy(acc.at[pl.ds(r * NN, NN)], s_out.at[r, d],
                             semw)
    for r in range(RR):
        pltpu.make_async_copy(acc.at[pl.ds(r * NN, NN)],
                              s_out.at[r, wid * DPT + DPT - 1], semw).wait()

    if make_counts:
        @pl.when(wid < NCNT)
        def _():
            zero_acc()
            for j in range(CNT_CHUNKS):
                pltpu.sync_copy(comb2.at[wid * CNT_CHUNKS + j], cbuf0)

                @plsc.parallel_loop(0, CE // 16, unroll=16)
                def _(jj):
                    c16 = cbuf0[pl.ds(jj * 16, 16)]
                    a16 = jnp.right_shift(c16, SRC_BITS)
                    plsc.addupdate_scatter(acc, [a16],
                                           jnp.ones((16,), jnp.float32))
            for r in range(RR):
                pltpu.sync_copy(acc.at[pl.ds(r * NN, NN)], cnt_out.at[r, wid])


def _make_sc_layer(make_counts):
    out_type = [jax.ShapeDtypeStruct((RR, DD, NN), jnp.float32)]
    if make_counts:
        out_type.append(jax.ShapeDtypeStruct((RR, NCNT, NN), jnp.float32))
    mesh = plsc.VectorSubcoreMesh(core_axis_name="c", subcore_axis_name="s")
    return pl.kernel(
        functools.partial(_sc_body, make_counts),
        out_type=tuple(out_type),
        mesh=mesh,
        compiler_params=pltpu.CompilerParams(
            needs_layout_passes=False, use_tc_tiling_on_sc=False),
        scratch_types=[
            pltpu.VMEM((NN,), jnp.float32),
            pltpu.VMEM((NN,), jnp.float32),
            pltpu.VMEM((RN,), jnp.float32),
            pltpu.VMEM((CE,), jnp.int32),
            pltpu.VMEM((CE,), jnp.int32),
            pltpu.SemaphoreType.DMA,
            pltpu.SemaphoreType.DMA,
            pltpu.SemaphoreType.DMA,
            pltpu.SemaphoreType.DMA,
        ],
    )


_sc_layer_with_counts = _make_sc_layer(True)
_sc_layer = _make_sc_layer(False)


def _tc_body(final, xt_ref, s_ref, cnt_ref, wrelT_ref, wrootT_ref, b_ref,
             out_ref, *rest):
    r = pl.program_id(0)
    if final:
        acc_ref, = rest
        invc = cnt_ref[0]
    else:
        invc_out_ref, acc_ref = rest
        cnt_r = jnp.sum(cnt_ref[0], axis=0, keepdims=True)
        invc = 1.0 / jnp.maximum(cnt_r, 1.0)
        invc_out_ref[0] = invc
    m = s_ref[0] * invc
    part = jnp.dot(wrelT_ref[0], m,
                   preferred_element_type=jnp.float32)

    @pl.when(r == 0)
    def _():
        root = jnp.dot(wrootT_ref[...], xt_ref[...],
                       preferred_element_type=jnp.float32)
        acc_ref[...] = root + b_ref[...]

    acc_ref[...] += part

    @pl.when(r == RR - 1)
    def _():
        res = jnp.maximum(acc_ref[...], 0.0)
        if final:
            out_ref[...] = res.T
        else:
            out_ref[...] = res


def _make_tc_layer(final):
    if final:
        out_specs = pl.BlockSpec((NN, DD), lambda r: (0, 0))
        out_shape = jax.ShapeDtypeStruct((NN, DD), jnp.float32)
        cnt_spec = pl.BlockSpec((1, 1, NN), lambda r: (r, 0, 0))
    else:
        out_specs = (
            pl.BlockSpec((DD, NN), lambda r: (0, 0)),
            pl.BlockSpec((1, 1, NN), lambda r: (r, 0, 0)),
        )
        out_shape = (
            jax.ShapeDtypeStruct((DD, NN), jnp.float32),
            jax.ShapeDtypeStruct((RR, 1, NN), jnp.float32),
        )
        cnt_spec = pl.BlockSpec((1, NCNT, NN), lambda r: (r, 0, 0))
    return pl.pallas_call(
        functools.partial(_tc_body, final),
        grid=(RR,),
        in_specs=[
            pl.BlockSpec((DD, NN), lambda r: (0, 0)),
            pl.BlockSpec((1, DD, NN), lambda r: (r, 0, 0)),
            cnt_spec,
            pl.BlockSpec((1, DD, DD), lambda r: (r, 0, 0)),
            pl.BlockSpec((DD, DD), lambda r: (0, 0)),
            pl.BlockSpec((DD, 1), lambda r: (0, 0)),
        ],
        out_specs=out_specs,
        out_shape=out_shape,
        scratch_shapes=[pltpu.VMEM((DD, NN), jnp.float32)],
    )


_tc_mid = _make_tc_layer(False)
_tc_final = _make_tc_layer(True)


def _tr_body(x_ref, o_ref):
    o_ref[...] = x_ref[...].T


_transpose_in = pl.pallas_call(
    _tr_body,
    in_specs=[pl.BlockSpec((NN, DD), lambda: (0, 0))],
    out_specs=pl.BlockSpec((DD, NN), lambda: (0, 0)),
    out_shape=jax.ShapeDtypeStruct((DD, NN), jnp.float32),
)


def kernel(x, edge_index, edge_type, Wrel0, Wroot0, b0, Wrel1, Wroot1, b1):
    src = edge_index[0].astype(jnp.int32)
    dst = edge_index[1].astype(jnp.int32)
    et = edge_type.astype(jnp.int32)
    aidx = et * NN + dst
    comb2 = ((aidx << SRC_BITS) | src).reshape(NCHUNK, CE)

    wrel0T = Wrel0.transpose(0, 2, 1)
    wrel1T = Wrel1.transpose(0, 2, 1)
    wroot0T = Wroot0.T
    wroot1T = Wroot1.T
    b0c = b0.reshape(DD, 1)
    b1c = b1.reshape(DD, 1)

    xt = _transpose_in(x)
    s1, cnt = _sc_layer_with_counts(xt, comb2)
    ht, invc = _tc_mid(xt, s1, cnt, wrel0T, wroot0T, b0c)
    (s2,) = _sc_layer(ht, comb2)
    out = _tc_final(ht, s2, invc, wrel1T, wroot1T, b1c)
    return out
